# baseline (speedup 1.0000x reference)
import functools

import jax
import jax.numpy as jnp
from jax.experimental import pallas as pl
from jax.experimental.pallas import tpu as pltpu

_NUM_CLASSES = 8
_CONF = 0.05
_IOU_T = 0.5
_MAX_DET = 100
_MAX_DET_PER_CLASS = 100

_N = 20000
_NPAD = 20480
_ROWS = 160
_LANES = 128


def _nms_kernel(x1, y1, x2, y2, cls, sc,
                sel_s, sel_y1, sel_x1, sel_y2, sel_x2,
                ny1, nx1, ny2, nx2, a2, s):
    p = pl.program_id(0)
    c_f = (p % _NUM_CLASSES).astype(jnp.float32)

    ny1v = y1[0] / 512.0
    nx1v = x1[0] / 512.0
    ny2v = y2[0] / 512.0
    nx2v = x2[0] / 512.0
    ny1[...] = ny1v
    nx1[...] = nx1v
    ny2[...] = ny2v
    nx2[...] = nx2v
    a2[...] = (ny2v - ny1v) * (nx2v - nx1v)
    s[...] = jnp.where((cls[0] == c_f) & (sc[0] > _CONF), sc[0], -jnp.inf)

    neg = jnp.full((1, _LANES), -jnp.inf, jnp.float32)
    zero = jnp.zeros((1, _LANES), jnp.float32)
    sel_s[0] = neg
    sel_y1[0] = zero
    sel_x1[0] = zero
    sel_y2[0] = zero
    sel_x2[0] = zero

    flatidx = (jax.lax.broadcasted_iota(jnp.int32, (_ROWS, _LANES), 0) * _LANES
               + jax.lax.broadcasted_iota(jnp.int32, (_ROWS, _LANES), 1))
    lane = jax.lax.broadcasted_iota(jnp.int32, (1, _LANES), 1)

    m0 = jnp.max(s[...])

    def cond(carry):
        step, m = carry
        return (step < _MAX_DET_PER_CLASS) & (m > _CONF)

    def body(carry):
        step, m = carry
        sv = s[...]
        eq = sv == m
        idx = jnp.min(jnp.where(eq, flatidx, jnp.int32(2 ** 30)))
        row = idx // _LANES
        col = idx % _LANES
        colmask = lane == col
        by1 = jnp.sum(jnp.where(colmask, ny1[pl.ds(row, 1), :], 0.0))
        bx1 = jnp.sum(jnp.where(colmask, nx1[pl.ds(row, 1), :], 0.0))
        by2 = jnp.sum(jnp.where(colmask, ny2[pl.ds(row, 1), :], 0.0))
        bx2 = jnp.sum(jnp.where(colmask, nx2[pl.ds(row, 1), :], 0.0))

        lm = lane == step
        sel_s[0] = jnp.where(lm, m, sel_s[0])
        sel_y1[0] = jnp.where(lm, by1, sel_y1[0])
        sel_x1[0] = jnp.where(lm, bx1, sel_x1[0])
        sel_y2[0] = jnp.where(lm, by2, sel_y2[0])
        sel_x2[0] = jnp.where(lm, bx2, sel_x2[0])

        yy1 = jnp.maximum(by1, ny1[...])
        xx1 = jnp.maximum(bx1, nx1[...])
        yy2 = jnp.minimum(by2, ny2[...])
        xx2 = jnp.minimum(bx2, nx2[...])
        inter = jnp.maximum(yy2 - yy1, 0.0) * jnp.maximum(xx2 - xx1, 0.0)
        a1 = (by2 - by1) * (bx2 - bx1)
        iou = inter / (a1 + a2[...] - inter + 1e-8)
        supp = (iou > _IOU_T) | (flatidx == idx)
        snew = jnp.where(supp, -jnp.inf, sv)
        s[...] = snew
        return step + 1, jnp.max(snew)

    jax.lax.while_loop(cond, body, (jnp.int32(0), m0))


def _merge_kernel(ms, my1, mx1, my2, mx2, res, scr):
    crow = jax.lax.broadcasted_iota(jnp.int32, (_NUM_CLASSES, _LANES), 0)
    lane = jax.lax.broadcasted_iota(jnp.int32, (_NUM_CLASSES, _LANES), 1)
    lane1 = jax.lax.broadcasted_iota(jnp.int32, (1, _LANES), 1)
    validlane = lane < _MAX_DET_PER_CLASS
    flat = jnp.where(validlane, crow * _MAX_DET_PER_CLASS + lane,
                     jnp.int32(2 ** 30))

    scr[...] = jnp.where(validlane, ms[0], -jnp.inf)
    res[0] = jnp.zeros((_NUM_CLASSES, _LANES), jnp.float32)

    m0 = jnp.max(scr[...])

    def cond(carry):
        step, m = carry
        return (step < _MAX_DET) & (m > _CONF)

    def body(carry):
        step, m = carry
        sv = scr[...]
        eq = sv == m
        fidx = jnp.min(jnp.where(eq, flat, jnp.int32(2 ** 30)))
        c = fidx // _MAX_DET_PER_CLASS
        j = fidx % _MAX_DET_PER_CLASS
        mask = (crow == c) & (lane == j)
        by1 = jnp.sum(jnp.where(mask, my1[0], 0.0))
        bx1 = jnp.sum(jnp.where(mask, mx1[0], 0.0))
        by2 = jnp.sum(jnp.where(mask, my2[0], 0.0))
        bx2 = jnp.sum(jnp.where(mask, mx2[0], 0.0))

        lm = lane1 == step
        res[0, pl.ds(0, 1), :] = jnp.where(lm, bx1 * 512.0, res[0, pl.ds(0, 1), :])
        res[0, pl.ds(1, 1), :] = jnp.where(lm, by1 * 512.0, res[0, pl.ds(1, 1), :])
        res[0, pl.ds(2, 1), :] = jnp.where(lm, bx2 * 512.0, res[0, pl.ds(2, 1), :])
        res[0, pl.ds(3, 1), :] = jnp.where(lm, by2 * 512.0, res[0, pl.ds(3, 1), :])
        res[0, pl.ds(4, 1), :] = jnp.where(lm, c.astype(jnp.float32), res[0, pl.ds(4, 1), :])
        res[0, pl.ds(5, 1), :] = jnp.where(lm, m, res[0, pl.ds(5, 1), :])

        snew = jnp.where(mask, -jnp.inf, sv)
        scr[...] = snew
        return step + 1, jnp.max(snew)

    nstep, _ = jax.lax.while_loop(cond, body, (jnp.int32(0), m0))
    res[0, pl.ds(6, 1), :] = jnp.where(lane1 == 0, nstep.astype(jnp.float32),
                                       res[0, pl.ds(6, 1), :])


@jax.jit
def kernel(images, predictions):
    B = predictions.shape[0]

    def _prep(a, pad_value):
        a = jnp.pad(a, ((0, 0), (0, _NPAD - _N)), constant_values=pad_value)
        return a.reshape(B, _ROWS, _LANES)

    x1 = _prep(predictions[..., 0], 0.0)
    y1 = _prep(predictions[..., 1], 0.0)
    x2 = _prep(predictions[..., 2], 0.0)
    y2 = _prep(predictions[..., 3], 0.0)
    cls = _prep(predictions[..., 4], -1.0)
    sc = _prep(predictions[..., 5], 0.0)

    nprog = B * _NUM_CLASSES
    in_spec = pl.BlockSpec((1, _ROWS, _LANES), lambda p: (p // _NUM_CLASSES, 0, 0))
    out_spec = pl.BlockSpec((1, 1, _LANES), lambda p: (p, 0, 0))
    out_shape = jax.ShapeDtypeStruct((nprog, 1, _LANES), jnp.float32)

    sel_s, sel_y1, sel_x1, sel_y2, sel_x2 = pl.pallas_call(
        _nms_kernel,
        grid=(nprog,),
        in_specs=[in_spec] * 6,
        out_specs=[out_spec] * 5,
        out_shape=[out_shape] * 5,
        scratch_shapes=[pltpu.VMEM((_ROWS, _LANES), jnp.float32)] * 6,
        compiler_params=pltpu.CompilerParams(
            dimension_semantics=("parallel",)),
    )(x1, y1, x2, y2, cls, sc)

    ms = sel_s.reshape(B, _NUM_CLASSES, _LANES)
    my1 = sel_y1.reshape(B, _NUM_CLASSES, _LANES)
    mx1 = sel_x1.reshape(B, _NUM_CLASSES, _LANES)
    my2 = sel_y2.reshape(B, _NUM_CLASSES, _LANES)
    mx2 = sel_x2.reshape(B, _NUM_CLASSES, _LANES)

    mspec = pl.BlockSpec((1, _NUM_CLASSES, _LANES), lambda b: (b, 0, 0))
    res = pl.pallas_call(
        _merge_kernel,
        grid=(B,),
        in_specs=[mspec] * 5,
        out_specs=mspec,
        out_shape=jax.ShapeDtypeStruct((B, _NUM_CLASSES, _LANES), jnp.float32),
        scratch_shapes=[pltpu.VMEM((_NUM_CLASSES, _LANES), jnp.float32)],
    )(ms, my1, mx1, my2, mx2)

    out6 = jnp.transpose(res[:, 0:6, 0:_MAX_DET], (0, 2, 1))
    valid_det = res[:, 6, 0].astype(jnp.int32)
    return out6, valid_det


# speedup vs baseline: 1.5155x; 1.5155x over previous
import functools

import jax
import jax.numpy as jnp
from jax.experimental import pallas as pl
from jax.experimental.pallas import tpu as pltpu

_NUM_CLASSES = 8
_CONF = 0.05
_IOU_T = 0.5
_MAX_DET = 100
_MAX_DET_PER_CLASS = 100

_N = 20000
_NPAD = 20480
_ROWS = 160
_LANES = 128


def _nms_kernel(x1, y1, x2, y2, cls, sc,
                sel_s, sel_y1, sel_x1, sel_y2, sel_x2,
                ny1, nx1, ny2, nx2, a2, s):
    ny1v = y1[0] / 512.0
    nx1v = x1[0] / 512.0
    ny2v = y2[0] / 512.0
    nx2v = x2[0] / 512.0
    ny1[...] = ny1v
    nx1[...] = nx1v
    ny2[...] = ny2v
    nx2[...] = nx2v
    a2[...] = (ny2v - ny1v) * (nx2v - nx1v)
    for c in range(_NUM_CLASSES):
        c_f = jnp.float32(c)
        s[c] = jnp.where((cls[0] == c_f) & (sc[0] > _CONF), sc[0], -jnp.inf)

    sel_s[0] = jnp.full((_NUM_CLASSES, _LANES), -jnp.inf, jnp.float32)
    zeros = jnp.zeros((_NUM_CLASSES, _LANES), jnp.float32)
    sel_y1[0] = zeros
    sel_x1[0] = zeros
    sel_y2[0] = zeros
    sel_x2[0] = zeros

    flatidx = (jax.lax.broadcasted_iota(jnp.int32, (_ROWS, _LANES), 0) * _LANES
               + jax.lax.broadcasted_iota(jnp.int32, (_ROWS, _LANES), 1))
    lane = jax.lax.broadcasted_iota(jnp.int32, (1, _LANES), 1)

    m_init = tuple(jnp.max(s[c]) for c in range(_NUM_CLASSES))

    def cond(carry):
        step = carry[0]
        ms = carry[1:]
        any_m = ms[0]
        for c in range(1, _NUM_CLASSES):
            any_m = jnp.maximum(any_m, ms[c])
        return (step < _MAX_DET_PER_CLASS) & (any_m > _CONF)

    def body(carry):
        step = carry[0]
        ms = carry[1:]
        new_ms = []
        for c in range(_NUM_CLASSES):
            m = ms[c]
            act = m > _CONF
            sv = s[c]
            eq = sv == m
            idx = jnp.min(jnp.where(eq, flatidx, jnp.int32(2 ** 30)))
            idx = jnp.where(act, idx, 0)
            row = idx // _LANES
            col = idx % _LANES
            colmask = lane == col
            by1 = jnp.sum(jnp.where(colmask, ny1[pl.ds(row, 1), :], 0.0))
            bx1 = jnp.sum(jnp.where(colmask, nx1[pl.ds(row, 1), :], 0.0))
            by2 = jnp.sum(jnp.where(colmask, ny2[pl.ds(row, 1), :], 0.0))
            bx2 = jnp.sum(jnp.where(colmask, nx2[pl.ds(row, 1), :], 0.0))

            lm = (lane == step) & act
            sel_s[0, pl.ds(c, 1), :] = jnp.where(lm, m, sel_s[0, pl.ds(c, 1), :])
            sel_y1[0, pl.ds(c, 1), :] = jnp.where(lm, by1, sel_y1[0, pl.ds(c, 1), :])
            sel_x1[0, pl.ds(c, 1), :] = jnp.where(lm, bx1, sel_x1[0, pl.ds(c, 1), :])
            sel_y2[0, pl.ds(c, 1), :] = jnp.where(lm, by2, sel_y2[0, pl.ds(c, 1), :])
            sel_x2[0, pl.ds(c, 1), :] = jnp.where(lm, bx2, sel_x2[0, pl.ds(c, 1), :])

            yy1 = jnp.maximum(by1, ny1[...])
            xx1 = jnp.maximum(bx1, nx1[...])
            yy2 = jnp.minimum(by2, ny2[...])
            xx2 = jnp.minimum(bx2, nx2[...])
            inter = jnp.maximum(yy2 - yy1, 0.0) * jnp.maximum(xx2 - xx1, 0.0)
            a1 = (by2 - by1) * (bx2 - bx1)
            iou = inter / (a1 + a2[...] - inter + 1e-8)
            snew = jnp.where(iou > _IOU_T, -jnp.inf, sv)
            s[c] = snew
            new_ms.append(jnp.max(snew))
        return (step + 1,) + tuple(new_ms)

    jax.lax.while_loop(cond, body, (jnp.int32(0),) + m_init)


def _merge_kernel(ms, my1, mx1, my2, mx2, res, scr):
    crow = jax.lax.broadcasted_iota(jnp.int32, (_NUM_CLASSES, _LANES), 0)
    lane = jax.lax.broadcasted_iota(jnp.int32, (_NUM_CLASSES, _LANES), 1)
    lane1 = jax.lax.broadcasted_iota(jnp.int32, (1, _LANES), 1)
    validlane = lane < _MAX_DET_PER_CLASS
    flat = jnp.where(validlane, crow * _MAX_DET_PER_CLASS + lane,
                     jnp.int32(2 ** 30))

    scr[...] = jnp.where(validlane, ms[0], -jnp.inf)
    res[0] = jnp.zeros((_NUM_CLASSES, _LANES), jnp.float32)

    m0 = jnp.max(scr[...])

    def cond(carry):
        step, m = carry
        return (step < _MAX_DET) & (m > _CONF)

    def body(carry):
        step, m = carry
        sv = scr[...]
        eq = sv == m
        fidx = jnp.min(jnp.where(eq, flat, jnp.int32(2 ** 30)))
        c = fidx // _MAX_DET_PER_CLASS
        j = fidx % _MAX_DET_PER_CLASS
        mask = (crow == c) & (lane == j)
        by1 = jnp.sum(jnp.where(mask, my1[0], 0.0))
        bx1 = jnp.sum(jnp.where(mask, mx1[0], 0.0))
        by2 = jnp.sum(jnp.where(mask, my2[0], 0.0))
        bx2 = jnp.sum(jnp.where(mask, mx2[0], 0.0))

        lm = lane1 == step
        res[0, pl.ds(0, 1), :] = jnp.where(lm, bx1 * 512.0, res[0, pl.ds(0, 1), :])
        res[0, pl.ds(1, 1), :] = jnp.where(lm, by1 * 512.0, res[0, pl.ds(1, 1), :])
        res[0, pl.ds(2, 1), :] = jnp.where(lm, bx2 * 512.0, res[0, pl.ds(2, 1), :])
        res[0, pl.ds(3, 1), :] = jnp.where(lm, by2 * 512.0, res[0, pl.ds(3, 1), :])
        res[0, pl.ds(4, 1), :] = jnp.where(lm, c.astype(jnp.float32), res[0, pl.ds(4, 1), :])
        res[0, pl.ds(5, 1), :] = jnp.where(lm, m, res[0, pl.ds(5, 1), :])

        snew = jnp.where(mask, -jnp.inf, sv)
        scr[...] = snew
        return step + 1, jnp.max(snew)

    nstep, _ = jax.lax.while_loop(cond, body, (jnp.int32(0), m0))
    res[0, pl.ds(6, 1), :] = jnp.where(lane1 == 0, nstep.astype(jnp.float32),
                                       res[0, pl.ds(6, 1), :])


@jax.jit
def kernel(images, predictions):
    B = predictions.shape[0]

    def _prep(a, pad_value):
        a = jnp.pad(a, ((0, 0), (0, _NPAD - _N)), constant_values=pad_value)
        return a.reshape(B, _ROWS, _LANES)

    x1 = _prep(predictions[..., 0], 0.0)
    y1 = _prep(predictions[..., 1], 0.0)
    x2 = _prep(predictions[..., 2], 0.0)
    y2 = _prep(predictions[..., 3], 0.0)
    cls = _prep(predictions[..., 4], -1.0)
    sc = _prep(predictions[..., 5], 0.0)

    in_spec = pl.BlockSpec((1, _ROWS, _LANES), lambda b: (b, 0, 0))
    out_spec = pl.BlockSpec((1, _NUM_CLASSES, _LANES), lambda b: (b, 0, 0))
    out_shape = jax.ShapeDtypeStruct((B, _NUM_CLASSES, _LANES), jnp.float32)

    sel_s, sel_y1, sel_x1, sel_y2, sel_x2 = pl.pallas_call(
        _nms_kernel,
        grid=(B,),
        in_specs=[in_spec] * 6,
        out_specs=[out_spec] * 5,
        out_shape=[out_shape] * 5,
        scratch_shapes=[pltpu.VMEM((_ROWS, _LANES), jnp.float32)] * 5
        + [pltpu.VMEM((_NUM_CLASSES, _ROWS, _LANES), jnp.float32)],
        compiler_params=pltpu.CompilerParams(
            dimension_semantics=("parallel",)),
    )(x1, y1, x2, y2, cls, sc)

    mspec = pl.BlockSpec((1, _NUM_CLASSES, _LANES), lambda b: (b, 0, 0))
    res = pl.pallas_call(
        _merge_kernel,
        grid=(B,),
        in_specs=[mspec] * 5,
        out_specs=mspec,
        out_shape=jax.ShapeDtypeStruct((B, _NUM_CLASSES, _LANES), jnp.float32),
        scratch_shapes=[pltpu.VMEM((_NUM_CLASSES, _LANES), jnp.float32)],
    )(sel_s, sel_y1, sel_x1, sel_y2, sel_x2)

    out6 = jnp.transpose(res[:, 0:6, 0:_MAX_DET], (0, 2, 1))
    valid_det = res[:, 6, 0].astype(jnp.int32)
    return out6, valid_det


# speedup vs baseline: 3.3605x; 2.2174x over previous
import functools

import jax
import jax.numpy as jnp
from jax.experimental import pallas as pl
from jax.experimental.pallas import tpu as pltpu

_NUM_CLASSES = 8
_CONF = 0.05
_IOU_T = 0.5
_MAX_DET = 100
_MAX_DET_PER_CLASS = 100

_N = 20000
_NPAD = 20480
_ROWS = 160
_LANES = 128


def _nms_kernel(x1, y1, x2, y2, cls, sc,
                sel_s, sel_y1, sel_x1, sel_y2, sel_x2,
                ny1, nx1, ny2, nx2, a2, *s):
    ny1v = y1[0] / 512.0
    nx1v = x1[0] / 512.0
    ny2v = y2[0] / 512.0
    nx2v = x2[0] / 512.0
    ny1[...] = ny1v
    nx1[...] = nx1v
    ny2[...] = ny2v
    nx2[...] = nx2v
    a2[...] = (ny2v - ny1v) * (nx2v - nx1v)
    for c in range(_NUM_CLASSES):
        c_f = jnp.float32(c)
        s[c][...] = jnp.where((cls[0] == c_f) & (sc[0] > _CONF), sc[0],
                              -jnp.inf)

    sel_s[0] = jnp.full((_NUM_CLASSES, _LANES), -jnp.inf, jnp.float32)
    zeros = jnp.zeros((_NUM_CLASSES, _LANES), jnp.float32)
    sel_y1[0] = zeros
    sel_x1[0] = zeros
    sel_y2[0] = zeros
    sel_x2[0] = zeros

    flatidx = (jax.lax.broadcasted_iota(jnp.int32, (_ROWS, _LANES), 0) * _LANES
               + jax.lax.broadcasted_iota(jnp.int32, (_ROWS, _LANES), 1))
    lane = jax.lax.broadcasted_iota(jnp.int32, (1, _LANES), 1)

    m_init = tuple(jnp.max(s[c][...]) for c in range(_NUM_CLASSES))

    def cond(carry):
        step = carry[0]
        ms = carry[1:]
        any_m = ms[0]
        for c in range(1, _NUM_CLASSES):
            any_m = jnp.maximum(any_m, ms[c])
        return (step < _MAX_DET_PER_CLASS) & (any_m > _CONF)

    def body(carry):
        step = carry[0]
        ms = carry[1:]
        C = _NUM_CLASSES
        act = [ms[c] > _CONF for c in range(C)]
        sv = [s[c][...] for c in range(C)]
        idx = [None] * C
        for c in range(C):
            eq = sv[c] == ms[c]
            idx[c] = jnp.min(jnp.where(eq, flatidx, jnp.int32(2 ** 30)))
        row, colmask = [None] * C, [None] * C
        for c in range(C):
            i = jnp.where(act[c], idx[c], 0)
            idx[c] = i
            row[c] = i // _LANES
            colmask[c] = lane == (i % _LANES)
        by1, bx1, by2, bx2 = [None] * C, [None] * C, [None] * C, [None] * C
        for c in range(C):
            by1[c] = jnp.sum(jnp.where(colmask[c], ny1[pl.ds(row[c], 1), :], 0.0))
            bx1[c] = jnp.sum(jnp.where(colmask[c], nx1[pl.ds(row[c], 1), :], 0.0))
            by2[c] = jnp.sum(jnp.where(colmask[c], ny2[pl.ds(row[c], 1), :], 0.0))
            bx2[c] = jnp.sum(jnp.where(colmask[c], nx2[pl.ds(row[c], 1), :], 0.0))
        for c in range(C):
            lm = (lane == step) & act[c]
            sel_s[0, pl.ds(c, 1), :] = jnp.where(lm, ms[c], sel_s[0, pl.ds(c, 1), :])
            sel_y1[0, pl.ds(c, 1), :] = jnp.where(lm, by1[c], sel_y1[0, pl.ds(c, 1), :])
            sel_x1[0, pl.ds(c, 1), :] = jnp.where(lm, bx1[c], sel_x1[0, pl.ds(c, 1), :])
            sel_y2[0, pl.ds(c, 1), :] = jnp.where(lm, by2[c], sel_y2[0, pl.ds(c, 1), :])
            sel_x2[0, pl.ds(c, 1), :] = jnp.where(lm, bx2[c], sel_x2[0, pl.ds(c, 1), :])
        snew = [None] * C
        for c in range(C):
            yy1 = jnp.maximum(by1[c], ny1[...])
            xx1 = jnp.maximum(bx1[c], nx1[...])
            yy2 = jnp.minimum(by2[c], ny2[...])
            xx2 = jnp.minimum(bx2[c], nx2[...])
            inter = jnp.maximum(yy2 - yy1, 0.0) * jnp.maximum(xx2 - xx1, 0.0)
            a1 = (by2[c] - by1[c]) * (bx2[c] - bx1[c])
            iou = inter / (a1 + a2[...] - inter + 1e-8)
            snew[c] = jnp.where(iou > _IOU_T, -jnp.inf, sv[c])
            s[c][...] = snew[c]
        new_ms = [jnp.max(snew[c]) for c in range(C)]
        return (step + 1,) + tuple(new_ms)

    jax.lax.while_loop(cond, body, (jnp.int32(0),) + m_init)


def _merge_kernel(ms, my1, mx1, my2, mx2, res, scr):
    crow = jax.lax.broadcasted_iota(jnp.int32, (_NUM_CLASSES, _LANES), 0)
    lane = jax.lax.broadcasted_iota(jnp.int32, (_NUM_CLASSES, _LANES), 1)
    lane1 = jax.lax.broadcasted_iota(jnp.int32, (1, _LANES), 1)
    validlane = lane < _MAX_DET_PER_CLASS
    flat = jnp.where(validlane, crow * _MAX_DET_PER_CLASS + lane,
                     jnp.int32(2 ** 30))

    scr[...] = jnp.where(validlane, ms[0], -jnp.inf)
    res[0] = jnp.zeros((_NUM_CLASSES, _LANES), jnp.float32)

    m0 = jnp.max(scr[...])

    def cond(carry):
        step, m = carry
        return (step < _MAX_DET) & (m > _CONF)

    def body(carry):
        step, m = carry
        sv = scr[...]
        eq = sv == m
        fidx = jnp.min(jnp.where(eq, flat, jnp.int32(2 ** 30)))
        c = fidx // _MAX_DET_PER_CLASS
        j = fidx % _MAX_DET_PER_CLASS
        mask = (crow == c) & (lane == j)
        by1 = jnp.sum(jnp.where(mask, my1[0], 0.0))
        bx1 = jnp.sum(jnp.where(mask, mx1[0], 0.0))
        by2 = jnp.sum(jnp.where(mask, my2[0], 0.0))
        bx2 = jnp.sum(jnp.where(mask, mx2[0], 0.0))

        lm = lane1 == step
        res[0, pl.ds(0, 1), :] = jnp.where(lm, bx1 * 512.0, res[0, pl.ds(0, 1), :])
        res[0, pl.ds(1, 1), :] = jnp.where(lm, by1 * 512.0, res[0, pl.ds(1, 1), :])
        res[0, pl.ds(2, 1), :] = jnp.where(lm, bx2 * 512.0, res[0, pl.ds(2, 1), :])
        res[0, pl.ds(3, 1), :] = jnp.where(lm, by2 * 512.0, res[0, pl.ds(3, 1), :])
        res[0, pl.ds(4, 1), :] = jnp.where(lm, c.astype(jnp.float32), res[0, pl.ds(4, 1), :])
        res[0, pl.ds(5, 1), :] = jnp.where(lm, m, res[0, pl.ds(5, 1), :])

        snew = jnp.where(mask, -jnp.inf, sv)
        scr[...] = snew
        return step + 1, jnp.max(snew)

    nstep, _ = jax.lax.while_loop(cond, body, (jnp.int32(0), m0))
    res[0, pl.ds(6, 1), :] = jnp.where(lane1 == 0, nstep.astype(jnp.float32),
                                       res[0, pl.ds(6, 1), :])


@jax.jit
def kernel(images, predictions):
    B = predictions.shape[0]

    def _prep(a, pad_value):
        a = jnp.pad(a, ((0, 0), (0, _NPAD - _N)), constant_values=pad_value)
        return a.reshape(B, _ROWS, _LANES)

    x1 = _prep(predictions[..., 0], 0.0)
    y1 = _prep(predictions[..., 1], 0.0)
    x2 = _prep(predictions[..., 2], 0.0)
    y2 = _prep(predictions[..., 3], 0.0)
    cls = _prep(predictions[..., 4], -1.0)
    sc = _prep(predictions[..., 5], 0.0)

    in_spec = pl.BlockSpec((1, _ROWS, _LANES), lambda b: (b, 0, 0))
    out_spec = pl.BlockSpec((1, _NUM_CLASSES, _LANES), lambda b: (b, 0, 0))
    out_shape = jax.ShapeDtypeStruct((B, _NUM_CLASSES, _LANES), jnp.float32)

    sel_s, sel_y1, sel_x1, sel_y2, sel_x2 = pl.pallas_call(
        _nms_kernel,
        grid=(B,),
        in_specs=[in_spec] * 6,
        out_specs=[out_spec] * 5,
        out_shape=[out_shape] * 5,
        scratch_shapes=[pltpu.VMEM((_ROWS, _LANES), jnp.float32)]
        * (5 + _NUM_CLASSES),
        compiler_params=pltpu.CompilerParams(
            dimension_semantics=("parallel",)),
    )(x1, y1, x2, y2, cls, sc)

    mspec = pl.BlockSpec((1, _NUM_CLASSES, _LANES), lambda b: (b, 0, 0))
    res = pl.pallas_call(
        _merge_kernel,
        grid=(B,),
        in_specs=[mspec] * 5,
        out_specs=mspec,
        out_shape=jax.ShapeDtypeStruct((B, _NUM_CLASSES, _LANES), jnp.float32),
        scratch_shapes=[pltpu.VMEM((_NUM_CLASSES, _LANES), jnp.float32)],
    )(sel_s, sel_y1, sel_x1, sel_y2, sel_x2)

    out6 = jnp.transpose(res[:, 0:6, 0:_MAX_DET], (0, 2, 1))
    valid_det = res[:, 6, 0].astype(jnp.int32)
    return out6, valid_det
